# baseline (speedup 1.0000x reference)
import functools

import jax
import jax.numpy as jnp
from jax import lax
from jax.experimental import pallas as pl
from jax.experimental.pallas import tpu as pltpu
from jax.experimental.pallas import tpu_sc as plsc

_B = 16384
_D = 32
_NC = 2
_NS = 16
_NW = _NC * _NS
_BPW = _B // _NW
_CH = 128
_NCH = _BPW // _CH


def _sc_gather(uid, iid, t_um, t_im, t_ug, t_ig):
    mesh = plsc.VectorSubcoreMesh(core_axis_name="c", subcore_axis_name="s")

    @functools.partial(
        pl.kernel,
        out_type=[jax.ShapeDtypeStruct((_B, _D), jnp.float32)] * 4,
        mesh=mesh,
        compiler_params=pltpu.CompilerParams(use_tc_tiling_on_sc=False),
        scratch_types=[
            pltpu.VMEM((_BPW,), jnp.int32),
            pltpu.VMEM((_BPW,), jnp.int32),
            pltpu.VMEM((_BPW, _D), jnp.float32),
            pltpu.VMEM((_BPW, _D), jnp.float32),
            pltpu.VMEM((_BPW, _D), jnp.float32),
            pltpu.VMEM((_BPW, _D), jnp.float32),
            pltpu.SemaphoreType.DMA,
        ],
    )
    def k(uid_hbm, iid_hbm, um_hbm, im_hbm, ug_hbm, ig_hbm,
          o_um, o_im, o_ug, o_ig,
          uid_v, iid_v, um_v, im_v, ug_v, ig_v, sem):
        wid = lax.axis_index("s") * _NC + lax.axis_index("c")
        base = wid * _BPW
        pltpu.sync_copy(uid_hbm.at[pl.ds(base, _BPW)], uid_v)
        pltpu.sync_copy(iid_hbm.at[pl.ds(base, _BPW)], iid_v)
        copies = []
        for c in range(_NCH):
            s = pl.ds(c * _CH, _CH)
            copies.append(pltpu.async_copy(um_hbm.at[uid_v.at[s]], um_v.at[s], sem))
            copies.append(pltpu.async_copy(im_hbm.at[iid_v.at[s]], im_v.at[s], sem))
            copies.append(pltpu.async_copy(ug_hbm.at[uid_v.at[s]], ug_v.at[s], sem))
            copies.append(pltpu.async_copy(ig_hbm.at[iid_v.at[s]], ig_v.at[s], sem))
        for cp in copies:
            cp.wait()
        pltpu.sync_copy(um_v, o_um.at[pl.ds(base, _BPW)])
        pltpu.sync_copy(im_v, o_im.at[pl.ds(base, _BPW)])
        pltpu.sync_copy(ug_v, o_ug.at[pl.ds(base, _BPW)])
        pltpu.sync_copy(ig_v, o_ig.at[pl.ds(base, _BPW)])

    return k(uid, iid, t_um, t_im, t_ug, t_ig)


def _dense_body(ue_ref, ie_ref, ug_ref, ig_ref, w1u_ref, w1i_ref, b1_ref,
                w2_ref, b2_ref, wh_ref, wg_ref, bo_ref, o_ref):
    h1 = jnp.dot(ue_ref[...], w1u_ref[...], preferred_element_type=jnp.float32)
    h1 = h1 + jnp.dot(ie_ref[...], w1i_ref[...], preferred_element_type=jnp.float32)
    h1 = jnp.maximum(h1 + b1_ref[...], 0.0)
    h2 = jnp.dot(h1, w2_ref[...], preferred_element_type=jnp.float32)
    h2 = jnp.maximum(h2 + b2_ref[...], 0.0)
    gmf = ug_ref[...] * ig_ref[...]
    logit = jnp.dot(h2, wh_ref[...], preferred_element_type=jnp.float32)
    logit = logit + jnp.dot(gmf, wg_ref[...], preferred_element_type=jnp.float32)
    o_ref[...] = logit + bo_ref[...]


def _tc_dense(ue, ie, ug, ig, w1u, w1i, b1, w2, b2, wh, wg, bo):
    bb = 2048
    grid = _B // bb
    row_spec = pl.BlockSpec((bb, _D), lambda i: (i, 0))

    def w_spec(shape):
        return pl.BlockSpec(shape, lambda i: (0,) * len(shape))

    return pl.pallas_call(
        _dense_body,
        grid=(grid,),
        in_specs=[
            row_spec, row_spec, row_spec, row_spec,
            w_spec((_D, 32)), w_spec((_D, 32)), w_spec((1, 32)),
            w_spec((32, 16)), w_spec((1, 16)),
            w_spec((16, 1)), w_spec((_D, 1)), w_spec((1, 1)),
        ],
        out_specs=pl.BlockSpec((bb, 1), lambda i: (i, 0)),
        out_shape=jax.ShapeDtypeStruct((_B, 1), jnp.float32),
    )(ue, ie, ug, ig, w1u, w1i, b1, w2, b2, wh, wg, bo)


def kernel(userID, itemID, user_emb_mlp, item_emb_mlp, user_emb_gmf,
           item_emb_gmf, W1, b1, W2, b2, Wout, bout):
    uid = userID.astype(jnp.int32)
    iid = itemID.astype(jnp.int32)
    ue, iem, ug, ig = _sc_gather(uid, iid, user_emb_mlp, item_emb_mlp,
                                 user_emb_gmf, item_emb_gmf)
    out = _tc_dense(ue, iem, ug, ig,
                    W1[:_D], W1[_D:], b1.reshape(1, -1),
                    W2, b2.reshape(1, -1),
                    Wout[:16], Wout[16:], bout.reshape(1, 1))
    return out.reshape(-1)
